# baseline (speedup 1.0000x reference)
import functools
import math

import jax
import jax.numpy as jnp
from jax import lax
from jax.experimental import pallas as pl
from jax.experimental.pallas import tpu as pltpu
from jax.experimental.pallas import tpu_sc as plsc

D = 128
SCALE = math.sqrt(D)

NC = 2
NS = 16
NW = NC * NS

CH = 128
NBUF = 6
A = NBUF // 2


@functools.partial(jax.jit, static_argnums=(2,))
def _gather(idx, lut, b_per_w):
    n_ch = b_per_w // CH
    n_grp = n_ch // NBUF
    mesh = plsc.VectorSubcoreMesh(core_axis_name="c", subcore_axis_name="s")

    @functools.partial(
        pl.kernel,
        out_type=jax.ShapeDtypeStruct((NW, n_ch, CH, D), jnp.float32),
        mesh=mesh,
        scratch_types=[
            pltpu.VMEM((n_ch, CH), jnp.int32),
            pltpu.VMEM((NBUF, CH, D), jnp.float32),
            [pltpu.SemaphoreType.DMA] * NBUF,
            [pltpu.SemaphoreType.DMA] * NBUF,
        ],
    )
    def body(idx_hbm, lut_hbm, out_hbm, idx_v, rows_v, gsems, ssems):
        wid = lax.axis_index("s") * NC + lax.axis_index("c")
        pltpu.sync_copy(idx_hbm.at[wid], idx_v)

        def scale_buf(b):
            def _row(r, _):
                for k in range(D // 16):
                    sl = pl.ds(k * 16, 16)
                    rows_v[b, r, sl] = rows_v[b, r, sl] * SCALE
                return ()

            lax.fori_loop(0, CH, _row, (), unroll=2)

        def start_gather(c, b):
            pltpu.async_copy(lut_hbm.at[idx_v.at[c]], rows_v.at[b], gsems[b])

        def wait_gather(c, b):
            pltpu.make_async_copy(
                lut_hbm.at[idx_v.at[c]], rows_v.at[b], gsems[b]
            ).wait()

        def start_scatter(c, b):
            pltpu.async_copy(rows_v.at[b], out_hbm.at[wid, c], ssems[b])

        def wait_scatter(c, b):
            pltpu.make_async_copy(
                rows_v.at[b], out_hbm.at[wid, c], ssems[b]
            ).wait()

        def finish(c, b):
            wait_gather(c, b)
            scale_buf(b)
            start_scatter(c, b)

        for c in range(NBUF):
            start_gather(c, c)
            if c >= A:
                finish(c - A, c - A)

        def grp(g, _):
            for b in range(NBUF):
                c = g * NBUF + b
                wait_scatter(c - NBUF, b)
                start_gather(c, b)
                finish(c - A, (b - A) % NBUF)
            return ()

        lax.fori_loop(1, n_grp, grp, (), unroll=False)

        for c in range(n_grp * NBUF, n_ch):
            b = c % NBUF
            wait_scatter(c - NBUF, b)
            start_gather(c, b)
            finish(c - A, (b - A) % NBUF)
        for cs in range(n_ch - A, n_ch):
            finish(cs, cs % NBUF)
        for cs in range(n_ch - NBUF, n_ch):
            wait_scatter(cs, cs % NBUF)

    return body(idx, lut)


def kernel(x, lut):
    s, t = x.shape
    b = s * t
    b_per_w = b // NW
    assert b_per_w % CH == 0
    idx = x.astype(jnp.int32).reshape(NW, b_per_w // CH, CH)
    out = _gather(idx, lut, b_per_w)
    return out.reshape(s, t, D)
